# baseline (speedup 1.0000x reference)
import functools
import jax
import jax.numpy as jnp
from jax import lax
from jax.experimental import pallas as pl
from jax.experimental.pallas import tpu as pltpu
from jax.experimental.pallas import tpu_sc as plsc

NC, NS, L = 2, 16, 16
NW = NC * NS
B, S, D, C = 4096, 50, 128, 100
SPW = B // NW
G = 2
NG = SPW // G
RPG = G * S
DC = D // L
NBUF = 4
MCHUNK = (0, 16, 32, 34)


def _pool_body(inp_hbm, mask_hbm, table_hbm, out_hbm,
               idx_v, mask_v, rows, sems, out_v):
    wid = lax.axis_index("s") * NC + lax.axis_index("c")
    pltpu.sync_copy(inp_hbm.at[wid], idx_v)
    pltpu.sync_copy(mask_hbm.at[pl.ds(wid * SPW, SPW)], mask_v)

    def start(g, bsel):
        pltpu.async_copy(table_hbm.at[idx_v.at[g]], rows[bsel], sems[bsel])

    for bsel in range(NBUF):
        start(bsel, bsel)

    def group(g, bsel):
        rb, sb = rows[bsel], sems[bsel]
        pltpu.make_async_copy(table_hbm.at[idx_v.at[0]], rb, sb).wait()
        obase = g * (G * D)

        def sample(n, _):
            samp = g * G + n
            mrows = [mask_v[samp, pl.ds(MCHUNK[k], L)] for k in range(4)]
            rbase = n * S
            for half in range(2):
                accs = [jnp.zeros((L,), jnp.float32) for _ in range(4)]
                for s in range(S):
                    k = 3 if s >= 48 else s // L
                    mv = jnp.full((L,), mrows[k][s - MCHUNK[k]], jnp.float32)
                    r = rbase + s
                    for c4 in range(4):
                        accs[c4] = accs[c4] + mv * rb[
                            r, pl.ds((half * 4 + c4) * L, L)]
                for c4 in range(4):
                    out_v[pl.ds(obase + n * D + (half * 4 + c4) * L, L)] = (
                        accs[c4])
            return 0

        lax.fori_loop(0, G, sample, 0)

    def outer(gq, carry):
        for bsel in range(NBUF):
            g = gq * NBUF + bsel
            group(g, bsel)

            @pl.when(g + NBUF < NG)
            def _():
                start(g + NBUF, bsel)
        return carry

    lax.fori_loop(0, NG // NBUF, outer, 0)
    pltpu.sync_copy(out_v, out_hbm.at[pl.ds(wid * SPW * D, SPW * D)])


@functools.partial(
    pl.kernel,
    out_type=jax.ShapeDtypeStruct((B * D,), jnp.float32),
    mesh=plsc.VectorSubcoreMesh(
        core_axis_name="c", subcore_axis_name="s",
        num_cores=NC, num_subcores=NS),
    scratch_types=[
        pltpu.VMEM((NG, RPG), jnp.int32),
        pltpu.VMEM((SPW, S), jnp.float32),
    ] + [pltpu.VMEM((RPG, D), jnp.float32) for _ in range(NBUF)]
      + [pltpu.VMEM((SPW * D,), jnp.float32)]
      + [pltpu.SemaphoreType.DMA for _ in range(NBUF)],
)
def _pool(inp_hbm, mask_hbm, table_hbm, out_hbm,
          idx_v, mask_v, r0, r1, r2, r3, out_v, s0, s1, s2, s3):
    _pool_body(inp_hbm, mask_hbm, table_hbm, out_hbm,
               idx_v, mask_v, (r0, r1, r2, r3), (s0, s1, s2, s3), out_v)


def _head_body(avg_ref, w_ref, b_ref, out_ref):
    x = jnp.dot(avg_ref[...], w_ref[...],
                preferred_element_type=jnp.float32)
    z = x + b_ref[...]
    out_ref[...] = 1.0 / (1.0 + jnp.exp(-z))


def _head(avg, w_s, b_s):
    return pl.pallas_call(
        _head_body,
        out_shape=jax.ShapeDtypeStruct((B, C), jnp.float32),
    )(avg, w_s, b_s)


def kernel(input, mask, embeddings, W, b):
    inp_r = input.astype(jnp.int32).reshape(NW, NG, RPG)
    sums = _pool(inp_r, mask.astype(jnp.float32), embeddings).reshape(B, D)
    w_s = (W.T * (1.0 / S)).astype(jnp.float32)
    b_s = b.astype(jnp.float32).reshape(1, C)
    return _head(sums, w_s, b_s)


# speedup vs baseline: 1.1468x; 1.1468x over previous
import functools
import jax
import jax.numpy as jnp
from jax import lax
from jax.experimental import pallas as pl
from jax.experimental.pallas import tpu as pltpu
from jax.experimental.pallas import tpu_sc as plsc

NC, NS, L = 2, 16, 16
NW = NC * NS
B, S, D, C = 4096, 50, 128, 100
SPW = B // NW
G = 2
NG = SPW // G
RPG = G * S
DC = D // L
NBUF = 4
MCHUNK = (0, 16, 32, 34)


def _pool_body(inp_hbm, mask_hbm, table_hbm, out_hbm,
               idx_v, mask_v, rows, sems, out_v):
    wid = lax.axis_index("s") * NC + lax.axis_index("c")
    pltpu.sync_copy(inp_hbm.at[wid], idx_v)
    pltpu.sync_copy(mask_hbm.at[pl.ds(wid * SPW, SPW)], mask_v)

    def start(g, bsel):
        pltpu.async_copy(table_hbm.at[idx_v.at[g]], rows[bsel], sems[bsel])

    for bsel in range(NBUF):
        start(bsel, bsel)

    def group(g, bsel):
        rb, sb = rows[bsel], sems[bsel]
        pltpu.make_async_copy(table_hbm.at[idx_v.at[0]], rb, sb).wait()
        obase = g * (G * D)

        def sample(n, _):
            samp = g * G + n
            mrows = [mask_v[samp, pl.ds(MCHUNK[k], L)] for k in range(4)]
            rbase = n * S
            for q in range(4):
                accs = [jnp.zeros((L,), jnp.float32) for _ in range(2)]
                for s in range(S):
                    k = 3 if s >= 48 else s // L
                    mv = jnp.full((L,), mrows[k][s - MCHUNK[k]], jnp.float32)
                    r = rbase + s
                    for c2 in range(2):
                        accs[c2] = accs[c2] + mv * rb[
                            r, pl.ds((q * 2 + c2) * L, L)]
                for c2 in range(2):
                    out_v[pl.ds(obase + n * D + (q * 2 + c2) * L, L)] = (
                        accs[c2])
            return 0

        lax.fori_loop(0, G, sample, 0)

    def outer(gq, carry):
        for bsel in range(NBUF):
            g = gq * NBUF + bsel
            group(g, bsel)

            @pl.when(g + NBUF < NG)
            def _():
                start(g + NBUF, bsel)
        return carry

    lax.fori_loop(0, NG // NBUF, outer, 0)
    pltpu.sync_copy(out_v, out_hbm.at[pl.ds(wid * SPW * D, SPW * D)])


@functools.partial(
    pl.kernel,
    out_type=jax.ShapeDtypeStruct((B * D,), jnp.float32),
    mesh=plsc.VectorSubcoreMesh(
        core_axis_name="c", subcore_axis_name="s",
        num_cores=NC, num_subcores=NS),
    scratch_types=[
        pltpu.VMEM((NG, RPG), jnp.int32),
        pltpu.VMEM((SPW, S), jnp.float32),
    ] + [pltpu.VMEM((RPG, D), jnp.float32) for _ in range(NBUF)]
      + [pltpu.VMEM((SPW * D,), jnp.float32)]
      + [pltpu.SemaphoreType.DMA for _ in range(NBUF)],
)
def _pool(inp_hbm, mask_hbm, table_hbm, out_hbm,
          idx_v, mask_v, r0, r1, r2, r3, out_v, s0, s1, s2, s3):
    _pool_body(inp_hbm, mask_hbm, table_hbm, out_hbm,
               idx_v, mask_v, (r0, r1, r2, r3), (s0, s1, s2, s3), out_v)


def _head_body(avg_ref, w_ref, b_ref, out_ref):
    x = jnp.dot(avg_ref[...], w_ref[...],
                preferred_element_type=jnp.float32)
    z = x + b_ref[...]
    out_ref[...] = 1.0 / (1.0 + jnp.exp(-z))


def _head(avg, w_s, b_s):
    return pl.pallas_call(
        _head_body,
        out_shape=jax.ShapeDtypeStruct((B, C), jnp.float32),
    )(avg, w_s, b_s)


def kernel(input, mask, embeddings, W, b):
    inp_r = input.astype(jnp.int32).reshape(NW, NG, RPG)
    sums = _pool(inp_r, mask.astype(jnp.float32), embeddings).reshape(B, D)
    w_s = (W.T * (1.0 / S)).astype(jnp.float32)
    b_s = b.astype(jnp.float32).reshape(1, C)
    return _head(sums, w_s, b_s)


# speedup vs baseline: 1.1693x; 1.0196x over previous
import functools
import jax
import jax.numpy as jnp
from jax import lax
from jax.experimental import pallas as pl
from jax.experimental.pallas import tpu as pltpu
from jax.experimental.pallas import tpu_sc as plsc

NC, NS, L = 2, 16, 16
NW = NC * NS
B, S, D, C = 4096, 50, 128, 100
SPW = B // NW
G = 2
NG = SPW // G
RPG = G * S
DC = D // L
NBUF = 4
MCHUNK = (0, 16, 32, 34)


def _pool_body(inp_hbm, mask_hbm, table_hbm, out_hbm,
               idx_v, mask_v, rows, sems, out_v):
    wid = lax.axis_index("s") * NC + lax.axis_index("c")
    pltpu.sync_copy(inp_hbm.at[wid], idx_v)

    def start(g, bsel):
        pltpu.async_copy(table_hbm.at[idx_v.at[g]], rows[bsel], sems[bsel])

    for bsel in range(NBUF):
        start(bsel, bsel)
    pltpu.sync_copy(mask_hbm.at[pl.ds(wid * SPW, SPW)], mask_v)

    def group(g, bsel):
        rb, sb = rows[bsel], sems[bsel]
        pltpu.make_async_copy(table_hbm.at[idx_v.at[0]], rb, sb).wait()
        obase = g * (G * D)

        def sample(n, _):
            samp = g * G + n
            mrows = [mask_v[samp, pl.ds(MCHUNK[k], L)] for k in range(4)]
            rbase = n * S
            for q in range(4):
                accs = [jnp.zeros((L,), jnp.float32) for _ in range(2)]
                for s in range(S):
                    k = 3 if s >= 48 else s // L
                    mv = jnp.full((L,), mrows[k][s - MCHUNK[k]], jnp.float32)
                    r = rbase + s
                    for c2 in range(2):
                        accs[c2] = accs[c2] + mv * rb[
                            r, pl.ds((q * 2 + c2) * L, L)]
                for c2 in range(2):
                    out_v[pl.ds(obase + n * D + (q * 2 + c2) * L, L)] = (
                        accs[c2])
            return 0

        lax.fori_loop(0, G, sample, 0)

    def outer(gq, carry):
        for bsel in range(NBUF):
            g = gq * NBUF + bsel
            group(g, bsel)

            @pl.when(g + NBUF < NG)
            def _():
                start(g + NBUF, bsel)
        return carry

    lax.fori_loop(0, NG // NBUF, outer, 0)
    pltpu.sync_copy(out_v, out_hbm.at[pl.ds(wid * SPW * D, SPW * D)])


@functools.partial(
    pl.kernel,
    out_type=jax.ShapeDtypeStruct((B * D,), jnp.float32),
    mesh=plsc.VectorSubcoreMesh(
        core_axis_name="c", subcore_axis_name="s",
        num_cores=NC, num_subcores=NS),
    scratch_types=[
        pltpu.VMEM((NG, RPG), jnp.int32),
        pltpu.VMEM((SPW, S), jnp.float32),
    ] + [pltpu.VMEM((RPG, D), jnp.float32) for _ in range(NBUF)]
      + [pltpu.VMEM((SPW * D,), jnp.float32)]
      + [pltpu.SemaphoreType.DMA for _ in range(NBUF)],
)
def _pool(inp_hbm, mask_hbm, table_hbm, out_hbm,
          idx_v, mask_v, r0, r1, r2, r3, out_v, s0, s1, s2, s3):
    _pool_body(inp_hbm, mask_hbm, table_hbm, out_hbm,
               idx_v, mask_v, (r0, r1, r2, r3), (s0, s1, s2, s3), out_v)


def _head_body(avg_ref, w_ref, b_ref, out_ref):
    x = jnp.dot(avg_ref[...], w_ref[...],
                preferred_element_type=jnp.float32)
    z = x + b_ref[...]
    out_ref[...] = 1.0 / (1.0 + jnp.exp(-z))


def _head(avg, w_s, b_s):
    return pl.pallas_call(
        _head_body,
        out_shape=jax.ShapeDtypeStruct((B, C), jnp.float32),
    )(avg, w_s, b_s)


def kernel(input, mask, embeddings, W, b):
    inp_r = input.astype(jnp.int32).reshape(NW, NG, RPG)
    sums = _pool(inp_r, mask.astype(jnp.float32), embeddings).reshape(B, D)
    w_s = (W.T * (1.0 / S)).astype(jnp.float32)
    b_s = b.astype(jnp.float32).reshape(1, C)
    return _head(sums, w_s, b_s)
